# baseline (speedup 1.0000x reference)
import functools
import jax
import jax.numpy as jnp
from jax import lax
from jax.experimental import pallas as pl
from jax.experimental.pallas import tpu as pltpu
from jax.experimental.pallas import tpu_sc as plsc

_NUM_EXPERTS = 64
_TOKEN_BLOCK = 4096

_NC = 2
_NS = 16
_NW = _NC * _NS
_LANES = 16
_SC_CHUNK = 256


def _softmax_block(x_ref, w_ref, probs_ref):
    x = x_ref[...]
    w = w_ref[...]
    logits = jnp.dot(x, w, preferred_element_type=jnp.float32)
    m = jnp.max(logits, axis=-1, keepdims=True)
    e = jnp.exp(logits - m)
    probs_ref[...] = e / jnp.sum(e, axis=-1, keepdims=True)


def _tc_softmax(hidden_states, router_weight):
    tokens, d_model = hidden_states.shape
    num_experts = router_weight.shape[1]
    block = _TOKEN_BLOCK
    return pl.pallas_call(
        _softmax_block,
        grid=(tokens // block,),
        in_specs=[
            pl.BlockSpec((block, d_model), lambda i: (i, 0)),
            pl.BlockSpec((d_model, num_experts), lambda i: (0, 0)),
        ],
        out_specs=pl.BlockSpec((block, num_experts), lambda i: (i, 0)),
        out_shape=jax.ShapeDtypeStruct((tokens, num_experts), jnp.float32),
    )(hidden_states, router_weight)


def _sc_route_body(pin_hbm, pout_hbm, map_hbm, pin_v, pout_v, map_v):
    wid = lax.axis_index("s") * _NC + lax.axis_index("c")
    words_per_worker = pin_hbm.shape[0] // _NW
    base = wid * words_per_worker
    chunk_words = _SC_CHUNK * _NUM_EXPERTS
    lane = lax.iota(jnp.int32, _LANES)
    neg_inf = jnp.full((_LANES,), -jnp.inf, jnp.float32)

    def chunk_body(c, _):
        word0 = base + c * chunk_words
        pltpu.sync_copy(pin_hbm.at[pl.ds(word0, chunk_words)], pin_v)

        def group_body(g, _):
            row0 = (g * _LANES + lane) * _NUM_EXPERTS

            def scan_expert(e, carry):
                m1, m2 = carry
                v = plsc.load_gather(pin_v, [row0 + e])
                m2 = jnp.maximum(m2, jnp.minimum(m1, v))
                m1 = jnp.maximum(m1, v)
                return m1, m2

            _, m2 = lax.fori_loop(0, _NUM_EXPERTS, scan_expert,
                                  (neg_inf, neg_inf))

            def write_expert(e, _):
                idx = row0 + e
                v = plsc.load_gather(pin_v, [idx])
                keep = v >= m2
                plsc.store_scatter(pout_v, [idx], jnp.where(keep, v, 0.0))
                plsc.store_scatter(map_v, [idx], jnp.where(keep, 1, 0))
                return 0

            lax.fori_loop(0, _NUM_EXPERTS, write_expert, 0)
            return 0

        lax.fori_loop(0, _SC_CHUNK // _LANES, group_body, 0)
        pltpu.sync_copy(pout_v, pout_hbm.at[pl.ds(word0, chunk_words)])
        pltpu.sync_copy(map_v, map_hbm.at[pl.ds(word0, chunk_words)])
        return 0

    lax.fori_loop(0, words_per_worker // chunk_words, chunk_body, 0)


def _sc_route(probs_flat):
    words = probs_flat.shape[0]
    chunk_words = _SC_CHUNK * _NUM_EXPERTS
    mesh = plsc.VectorSubcoreMesh(core_axis_name="c", subcore_axis_name="s")
    run = functools.partial(
        pl.kernel,
        out_type=[
            jax.ShapeDtypeStruct((words,), jnp.float32),
            jax.ShapeDtypeStruct((words,), jnp.int32),
        ],
        mesh=mesh,
        compiler_params=pltpu.CompilerParams(needs_layout_passes=False),
        scratch_types=[
            pltpu.VMEM((chunk_words,), jnp.float32),
            pltpu.VMEM((chunk_words,), jnp.float32),
            pltpu.VMEM((chunk_words,), jnp.int32),
        ],
    )(_sc_route_body)
    return run(probs_flat)


def kernel(hidden_states, router_weight):
    tokens, _ = hidden_states.shape
    num_experts = router_weight.shape[1]
    probs_full = _tc_softmax(hidden_states, router_weight)
    probs_flat, map_i32 = _sc_route(probs_full.reshape(-1))
    probs = probs_flat.reshape(tokens, num_experts)
    routing_map = map_i32.reshape(tokens, num_experts).astype(jnp.bool_)
    return probs, routing_map


# speedup vs baseline: 1.0855x; 1.0855x over previous
import functools
import jax
import jax.numpy as jnp
from jax import lax
from jax.experimental import pallas as pl
from jax.experimental.pallas import tpu as pltpu
from jax.experimental.pallas import tpu_sc as plsc

_NUM_EXPERTS = 64
_TOKEN_BLOCK = 4096

_NC = 2
_NS = 16
_NW = _NC * _NS
_LANES = 16
_SC_CHUNK = 256


def _softmax_block(x_ref, w_ref, probs_ref):
    x = x_ref[...]
    w = w_ref[...]
    logits = jnp.dot(x, w, preferred_element_type=jnp.float32)
    m = jnp.max(logits, axis=-1, keepdims=True)
    e = jnp.exp(logits - m)
    probs_ref[...] = e / jnp.sum(e, axis=-1, keepdims=True)


def _tc_softmax(hidden_states, router_weight):
    tokens, d_model = hidden_states.shape
    num_experts = router_weight.shape[1]
    block = _TOKEN_BLOCK
    return pl.pallas_call(
        _softmax_block,
        grid=(tokens // block,),
        in_specs=[
            pl.BlockSpec((block, d_model), lambda i: (i, 0)),
            pl.BlockSpec((d_model, num_experts), lambda i: (0, 0)),
        ],
        out_specs=pl.BlockSpec((block, num_experts), lambda i: (i, 0)),
        out_shape=jax.ShapeDtypeStruct((tokens, num_experts), jnp.float32),
    )(hidden_states, router_weight)


def _sc_route_body(pin_hbm, pout_hbm, map_hbm, pin_v, pout_v, map_v):
    wid = lax.axis_index("s") * _NC + lax.axis_index("c")
    words_per_worker = pin_hbm.shape[0] // _NW
    base = wid * words_per_worker
    chunk_words = _SC_CHUNK * _NUM_EXPERTS
    lane = lax.iota(jnp.int32, _LANES)
    neg_inf = jnp.full((_LANES,), -jnp.inf, jnp.float32)

    def chunk_body(c, _):
        word0 = base + c * chunk_words
        pltpu.sync_copy(pin_hbm.at[pl.ds(word0, chunk_words)], pin_v)

        def group_body(g, _):
            row0 = (g * _LANES + lane) * _NUM_EXPERTS

            m1 = neg_inf
            m2 = neg_inf
            for e in range(_NUM_EXPERTS):
                v = plsc.load_gather(pin_v, [row0 + e])
                m2 = jnp.maximum(m2, jnp.minimum(m1, v))
                m1 = jnp.maximum(m1, v)

            one = jnp.ones((_LANES,), jnp.int32)
            zero_i = jnp.zeros((_LANES,), jnp.int32)
            for e in range(_NUM_EXPERTS):
                v = plsc.load_gather(pin_v, [row0 + e])
                keep = v >= m2
                plsc.store_scatter(pout_v, [row0 + e],
                                   jnp.where(keep, v, 0.0))
                plsc.store_scatter(map_v, [row0 + e],
                                   jnp.where(keep, one, zero_i))
            return 0

        lax.fori_loop(0, _SC_CHUNK // _LANES, group_body, 0)
        pltpu.sync_copy(pout_v, pout_hbm.at[pl.ds(word0, chunk_words)])
        pltpu.sync_copy(map_v, map_hbm.at[pl.ds(word0, chunk_words)])
        return 0

    lax.fori_loop(0, words_per_worker // chunk_words, chunk_body, 0)


def _sc_route(probs_flat):
    words = probs_flat.shape[0]
    chunk_words = _SC_CHUNK * _NUM_EXPERTS
    mesh = plsc.VectorSubcoreMesh(core_axis_name="c", subcore_axis_name="s")
    run = functools.partial(
        pl.kernel,
        out_type=[
            jax.ShapeDtypeStruct((words,), jnp.float32),
            jax.ShapeDtypeStruct((words,), jnp.int32),
        ],
        mesh=mesh,
        compiler_params=pltpu.CompilerParams(needs_layout_passes=False),
        scratch_types=[
            pltpu.VMEM((chunk_words,), jnp.float32),
            pltpu.VMEM((chunk_words,), jnp.float32),
            pltpu.VMEM((chunk_words,), jnp.int32),
        ],
    )(_sc_route_body)
    return run(probs_flat)


def kernel(hidden_states, router_weight):
    tokens, _ = hidden_states.shape
    num_experts = router_weight.shape[1]
    probs_full = _tc_softmax(hidden_states, router_weight)
    probs_flat, map_i32 = _sc_route(probs_full.reshape(-1))
    probs = probs_flat.reshape(tokens, num_experts)
    routing_map = map_i32.reshape(tokens, num_experts).astype(jnp.bool_)
    return probs, routing_map


# speedup vs baseline: 3.9282x; 3.6187x over previous
import jax
import jax.numpy as jnp
from jax.experimental import pallas as pl

_TOKEN_BLOCK = 4096


def _router_block(x_ref, w_ref, probs_ref, map_ref):
    x = x_ref[...]
    w = w_ref[...]
    logits = jnp.dot(x, w, preferred_element_type=jnp.float32)
    m1 = jnp.max(logits, axis=-1, keepdims=True)
    lm = jnp.where(logits == m1, -jnp.inf, logits)
    m2 = jnp.max(lm, axis=-1, keepdims=True)
    rmap = logits >= m2
    e = jnp.exp(logits - m1)
    s = jnp.sum(e, axis=-1, keepdims=True)
    probs_ref[...] = jnp.where(rmap, e, 0.0) / s
    map_ref[...] = rmap


def kernel(hidden_states, router_weight):
    tokens, d_model = hidden_states.shape
    num_experts = router_weight.shape[1]
    block = _TOKEN_BLOCK
    grid = (tokens // block,)
    probs, routing_map = pl.pallas_call(
        _router_block,
        grid=grid,
        in_specs=[
            pl.BlockSpec((block, d_model), lambda i: (i, 0)),
            pl.BlockSpec((d_model, num_experts), lambda i: (0, 0)),
        ],
        out_specs=[
            pl.BlockSpec((block, num_experts), lambda i: (i, 0)),
            pl.BlockSpec((block, num_experts), lambda i: (i, 0)),
        ],
        out_shape=[
            jax.ShapeDtypeStruct((tokens, num_experts), jnp.float32),
            jax.ShapeDtypeStruct((tokens, num_experts), jnp.bool_),
        ],
    )(hidden_states, router_weight)
    return probs, routing_map
